# Initial kernel scaffold; baseline (speedup 1.0000x reference)
import jax
import jax.numpy as jnp
from jax.experimental import pallas as pl


def kernel(inputs, W_head, g_head, b_head, W_blk, g_blk, b_blk, W_fus, g_fus, b_fus, W_p1, bias_p1, g_p1, b_p1, W_p2, bias_p2, g_p2, b_p2, W_p3, bias_p3):
    raise NotImplementedError("write your pallas kernel here")


# speedup vs baseline: 14.4638x; 14.4638x over previous
import functools

import jax
import jax.numpy as jnp
from jax import lax
from jax.experimental import pallas as pl
from jax.experimental.pallas import tpu as pltpu
from jax.experimental.pallas import tpu_sc as plsc

KK = 16
NBLK = 7
_INTERPRET = False
_USE_SC = True

NC, NS = 2, 16
NW = NC * NS


def _topk_body(N, R, xk_ref, xr_ref, sq_ref, sqr_ref, idx_ref):
    b = pl.program_id(0)
    xc = xk_ref[0]
    xr = xr_ref[0]
    inner = -2.0 * jnp.dot(xr, xc, preferred_element_type=jnp.float32)
    sq_row = sqr_ref[0][0, :]
    sq_col = sq_ref[0][0, :]
    d = (sq_row[:, None] + inner) + sq_col[None, :]
    col = lax.broadcasted_iota(jnp.int32, (R, N), 1)
    kcol = lax.broadcasted_iota(jnp.int32, (R, KK), 1)
    idx_acc = jnp.zeros((R, KK), jnp.int32)
    big = jnp.int32(N)
    for t in range(KK):
        mn = jnp.min(d, axis=1, keepdims=True)
        am = jnp.min(jnp.where(d == mn, col, big), axis=1)
        idx_acc = jnp.where(kcol == t, am[:, None], idx_acc)
        d = jnp.where(col == am[:, None], jnp.float32(jnp.inf), d)
    idx_ref[0] = idx_acc + b * N


def _topk(xk, xt, sq, R=256):
    B, C, N = xk.shape
    R = min(R, N)
    return pl.pallas_call(
        functools.partial(_topk_body, N, R),
        grid=(B, N // R),
        in_specs=[
            pl.BlockSpec((1, C, N), lambda b, r: (b, 0, 0)),
            pl.BlockSpec((1, R, C), lambda b, r: (b, r, 0)),
            pl.BlockSpec((1, 1, N), lambda b, r: (b, 0, 0)),
            pl.BlockSpec((1, 1, R), lambda b, r: (b, 0, r)),
        ],
        out_specs=pl.BlockSpec((1, R, KK), lambda b, r: (b, r, 0)),
        out_shape=jax.ShapeDtypeStruct((B, N, KK), jnp.int32),
        interpret=_INTERPRET,
    )(xk, xt, sq, sq)


def _sc_gather_body(P, CPP, C, xt_hbm, idx_hbm, xd_hbm,
                    idx_v, rows_v, xi_v, xd_v, sem):
    wid = lax.axis_index("s") * NC + lax.axis_index("c")
    base_pt = wid * P
    nchunks = P // CPP
    ncc = (C + 15) // 16

    def chunk_body(c, carry):
        pbase = base_pt + c * CPP
        pltpu.sync_copy(idx_hbm.at[pl.ds(pbase * KK, CPP * KK)], idx_v)
        pltpu.sync_copy(xt_hbm.at[pl.ds(pbase, CPP)], xi_v)
        pltpu.async_copy(xt_hbm.at[idx_v], rows_v, sem).wait()

        def point_body(p, carry):
            for ch in range(ncc):
                xi = xi_v[p, pl.ds(ch * 16, 16)]
                for k in range(KK):
                    v = rows_v[p * KK + k, pl.ds(ch * 16, 16)]
                    xd_v[k, p, pl.ds(ch * 16, 16)] = v - xi
            return carry

        carry = lax.fori_loop(0, CPP, point_body, carry)
        for k in range(KK):
            pltpu.sync_copy(xd_v.at[k], xd_hbm.at[k, pl.ds(pbase, CPP)])
        return carry

    lax.fori_loop(0, nchunks, chunk_body, 0)


def _sc_gather_xd(xt_pad, idx_flat, C):
    BN = xt_pad.shape[0]
    P = BN // NW
    CPP = 8
    Cp = 16 * ((C + 15) // 16)
    mesh = plsc.VectorSubcoreMesh(core_axis_name="c", subcore_axis_name="s")
    fn = functools.partial(
        pl.kernel,
        out_type=jax.ShapeDtypeStruct((KK, BN, Cp), jnp.float32),
        mesh=mesh,
        scratch_types=[
            pltpu.VMEM((CPP * KK,), jnp.int32),
            pltpu.VMEM((CPP * KK, 128), jnp.float32),
            pltpu.VMEM((CPP, 128), jnp.float32),
            pltpu.VMEM((KK, CPP, Cp), jnp.float32),
            pltpu.SemaphoreType.DMA,
        ],
    )(functools.partial(_sc_gather_body, P, CPP, Cp))
    return fn(xt_pad, idx_flat)


def _gather_xd_jax(xt_pad, idx_flat, C):
    BN = xt_pad.shape[0]
    Cp = 16 * ((C + 15) // 16)
    g = xt_pad[idx_flat][:, :Cp].reshape(BN, KK, Cp)
    xi = xt_pad[:, :Cp][:, None, :]
    return jnp.transpose(g - xi, (1, 0, 2))


def _econv_body(C, Cp, R, x_ref, xd_ref, w_ref, sm_ref, yp_ref):
    xi = x_ref[0]
    W = w_ref[...]
    tile_xi = jnp.concatenate([xi] * KK, axis=1)
    xd_cat = jnp.concatenate(
        [xd_ref[k].T[:C, :] for k in range(KK)], axis=1)
    feat = jnp.concatenate([tile_xi, xd_cat], axis=0)
    y = jnp.dot(W, feat, preferred_element_type=jnp.float32)
    mx = y[:, :R]
    yp_ref[0, 0] = y[:, :R]
    for k in range(1, KK):
        yk = y[:, k * R:(k + 1) * R]
        yp_ref[0, k] = yk
        mx = jnp.maximum(mx, yk)
    sm_ref[0] = mx


def _econv(x, xd_g, W, R=256):
    B, C, N = x.shape
    CO = W.shape[0]
    Cp = xd_g.shape[2]
    R = min(R, N)
    NB = N // R
    return pl.pallas_call(
        functools.partial(_econv_body, C, Cp, R),
        grid=(B, NB),
        in_specs=[
            pl.BlockSpec((1, C, R), lambda b, r: (b, 0, r)),
            pl.BlockSpec((KK, R, Cp), lambda b, r: (0, b * NB + r, 0)),
            pl.BlockSpec(W.shape, lambda b, r: (0, 0)),
        ],
        out_specs=[
            pl.BlockSpec((1, CO, R), lambda b, r: (b, 0, r)),
            pl.BlockSpec((1, KK, CO, R), lambda b, r: (b, 0, 0, r)),
        ],
        out_shape=[
            jax.ShapeDtypeStruct((B, CO, N), jnp.float32),
            jax.ShapeDtypeStruct((B, KK, CO, N), jnp.float32),
        ],
        interpret=_INTERPRET,
    )(x, xd_g, W)


def _edge_block(x, W, gamma, beta, knn_ch, res):
    B, C, N = x.shape
    xt = jnp.transpose(x, (0, 2, 1))
    xk_t = xt[:, :, :knn_ch]
    sq = jnp.sum(xk_t * xk_t, axis=-1)[:, None, :]
    idx = _topk(x[:, :knn_ch, :], xk_t, sq)
    xt_pad = jnp.concatenate(
        [xt, jnp.zeros((B, N, 128 - C), jnp.float32)], axis=-1).reshape(B * N, 128)
    idx_flat = idx.reshape(-1)
    if _USE_SC:
        xd_g = _sc_gather_xd(xt_pad, idx_flat, C)
    else:
        xd_g = _gather_xd_jax(xt_pad, idx_flat, C)
    sm, yp = _econv(x, xd_g, W)
    M = B * N * KK
    mu = jnp.sum(yp, axis=(0, 1, 3)) * jnp.float32(1.0 / M)
    dv = yp - mu[None, None, :, None]
    var = jnp.sum(dv * dv, axis=(0, 1, 3)) * jnp.float32(1.0 / M)
    yn = (sm - mu[None, :, None]) / jnp.sqrt(var + 1e-5)[None, :, None]
    out = jnp.maximum(yn * gamma[None, :, None] + beta[None, :, None], 0.0)
    if res is not None:
        out = out + res
    return out


def _leaky(x):
    return jnp.where(x >= 0, x, 0.2 * x)


def _fusion_body(nfeat, *refs):
    j = pl.program_id(0) * pl.num_programs(1) + pl.program_id(1)
    feats = [refs[i][0] for i in range(nfeat)]
    w_ref = refs[nfeat]
    z_ref, s1_ref, s2_ref = refs[nfeat + 1], refs[nfeat + 2], refs[nfeat + 3]
    xin = jnp.concatenate(feats, axis=0)
    z = jnp.dot(w_ref[...], xin, preferred_element_type=jnp.float32)
    z_ref[0] = z

    @pl.when(j == 0)
    def _():
        s1_ref[...] = jnp.zeros_like(s1_ref)
        s2_ref[...] = jnp.zeros_like(s2_ref)

    s1_ref[...] += jnp.sum(z, axis=1)[None, :]
    s2_ref[...] += jnp.sum(z * z, axis=1)[None, :]


def _fusion(feats, W, TN=512):
    B, C, N = feats[0].shape
    CO = W.shape[0]
    nfeat = len(feats)
    in_specs = [pl.BlockSpec((1, C, TN), lambda b, jj: (b, 0, jj)) for _ in feats]
    in_specs.append(pl.BlockSpec(W.shape, lambda b, jj: (0, 0)))
    return pl.pallas_call(
        functools.partial(_fusion_body, nfeat),
        grid=(B, N // TN),
        in_specs=in_specs,
        out_specs=[
            pl.BlockSpec((1, CO, TN), lambda b, jj: (b, 0, jj)),
            pl.BlockSpec((1, CO), lambda b, jj: (0, 0)),
            pl.BlockSpec((1, CO), lambda b, jj: (0, 0)),
        ],
        out_shape=[
            jax.ShapeDtypeStruct((B, CO, N), jnp.float32),
            jax.ShapeDtypeStruct((1, CO), jnp.float32),
            jax.ShapeDtypeStruct((1, CO), jnp.float32),
        ],
        interpret=_INTERPRET,
    )(*feats, W)


def _pool_body(M, z_ref, s1_ref, s2_ref, x1_ref, x2_ref):
    j = pl.program_id(1)
    mu = s1_ref[...][0] / M
    var = s2_ref[...][0] / M - mu * mu
    zn = _leaky((z_ref[0] - mu[:, None]) / jnp.sqrt(var + 1e-5)[:, None])

    @pl.when(j == 0)
    def _():
        x1_ref[...] = jnp.full_like(x1_ref, -3e38)
        x2_ref[...] = jnp.zeros_like(x2_ref)

    x1_ref[...] = jnp.maximum(x1_ref[...], jnp.max(zn, axis=1)[None, None, :])
    x2_ref[...] += jnp.sum(zn, axis=1)[None, None, :]


def _pool(z, s1, s2, M, TN=512):
    B, CO, N = z.shape
    return pl.pallas_call(
        functools.partial(_pool_body, M),
        grid=(B, N // TN),
        in_specs=[
            pl.BlockSpec((1, CO, TN), lambda b, jj: (b, 0, jj)),
            pl.BlockSpec((1, CO), lambda b, jj: (0, 0)),
            pl.BlockSpec((1, CO), lambda b, jj: (0, 0)),
        ],
        out_specs=[
            pl.BlockSpec((1, 1, CO), lambda b, jj: (b, 0, 0)),
            pl.BlockSpec((1, 1, CO), lambda b, jj: (b, 0, 0)),
        ],
        out_shape=[
            jax.ShapeDtypeStruct((B, 1, CO), jnp.float32),
            jax.ShapeDtypeStruct((B, 1, CO), jnp.float32),
        ],
        interpret=_INTERPRET,
    )(z, s1, s2)


def _mlp_body(M, act_out, z_ref, s1_ref, s2_ref, w_ref, c_ref,
              o_ref, t1_ref, t2_ref):
    j = pl.program_id(0) * pl.num_programs(1) + pl.program_id(1)
    mu = s1_ref[...][0] / M
    var = s2_ref[...][0] / M - mu * mu
    zn = _leaky((z_ref[0] - mu[:, None]) / jnp.sqrt(var + 1e-5)[:, None])
    o = jnp.dot(w_ref[...], zn, preferred_element_type=jnp.float32)
    o = o + c_ref[0, 0][:, None]
    if act_out == "logsoftmax":
        mx = jnp.max(o, axis=0, keepdims=True)
        lse = jnp.log(jnp.sum(jnp.exp(o - mx), axis=0, keepdims=True))
        o_ref[0] = o - mx - lse
        t1_ref[...] = jnp.zeros_like(t1_ref)
        t2_ref[...] = jnp.zeros_like(t2_ref)
        return
    o_ref[0] = o

    @pl.when(j == 0)
    def _():
        t1_ref[...] = jnp.zeros_like(t1_ref)
        t2_ref[...] = jnp.zeros_like(t2_ref)

    t1_ref[...] += jnp.sum(o, axis=1)[None, :]
    t2_ref[...] += jnp.sum(o * o, axis=1)[None, :]


def _mlp_layer(z, s1, s2, W, cvec, M, act_out="none", TN=512):
    B, CI, N = z.shape
    CO = W.shape[0]
    return pl.pallas_call(
        functools.partial(_mlp_body, M, act_out),
        grid=(B, N // TN),
        in_specs=[
            pl.BlockSpec((1, CI, TN), lambda b, jj: (b, 0, jj)),
            pl.BlockSpec((1, CI), lambda b, jj: (0, 0)),
            pl.BlockSpec((1, CI), lambda b, jj: (0, 0)),
            pl.BlockSpec(W.shape, lambda b, jj: (0, 0)),
            pl.BlockSpec((1, 1, CO), lambda b, jj: (b, 0, 0)),
        ],
        out_specs=[
            pl.BlockSpec((1, CO, TN), lambda b, jj: (b, 0, jj)),
            pl.BlockSpec((1, CO), lambda b, jj: (0, 0)),
            pl.BlockSpec((1, CO), lambda b, jj: (0, 0)),
        ],
        out_shape=[
            jax.ShapeDtypeStruct((B, CO, N), jnp.float32),
            jax.ShapeDtypeStruct((1, CO), jnp.float32),
            jax.ShapeDtypeStruct((1, CO), jnp.float32),
        ],
        interpret=_INTERPRET,
    )(z, s1, s2, W, cvec[:, None, :])


def kernel(inputs, W_head, g_head, b_head, W_blk, g_blk, b_blk,
           W_fus, g_fus, b_fus, W_p1, bias_p1, g_p1, b_p1,
           W_p2, bias_p2, g_p2, b_p2, W_p3, bias_p3):
    x0 = inputs[..., 0]
    B, _, N = x0.shape

    feats = [_edge_block(x0, W_head, g_head, b_head, knn_ch=3, res=None)]
    for i in range(NBLK - 1):
        x = feats[-1]
        feats.append(_edge_block(x, W_blk[i], g_blk[i], b_blk[i],
                                 knn_ch=x.shape[1], res=x))

    z, s1, s2 = _fusion(feats, W_fus)
    M = B * N
    x1, x2s = _pool(z, s1, s2, M)
    gp = jnp.concatenate([x1[:, 0], x2s[:, 0] / N], axis=1)

    Wg, Wf = W_p1[:, :2048], W_p1[:, 2048:]
    c1 = gp @ Wg.T + bias_p1[None, :]
    z1, t11, t12 = _mlp_layer(z, s1, s2, Wf, c1, M)
    c2 = jnp.broadcast_to(bias_p2[None, :], (B, W_p2.shape[0]))
    z2, t21, t22 = _mlp_layer(z1, t11, t12, W_p2, c2, M)
    c3 = jnp.broadcast_to(bias_p3[None, :], (B, W_p3.shape[0]))
    out, _, _ = _mlp_layer(z2, t21, t22, W_p3, c3, M, act_out="logsoftmax")
    return out
